# baseline (speedup 1.0000x reference)
import functools

import jax
import jax.numpy as jnp
from jax import lax
from jax.experimental import pallas as pl
from jax.experimental.pallas import tpu as pltpu
from jax.experimental.pallas import tpu_sc as plsc

EMB = 64
GROUP = 5
LN_EPS = 1e-12

NC, NS = 2, 16
NW = NC * NS
CH = 128
SUB = 8
SUBROWS = CH // SUB
NSLOT = 5
GAHEAD = 3


def _sc_gather(table, idx3):
    nchunk = idx3.shape[1]
    total = NW * nchunk * CH

    @functools.partial(
        pl.kernel,
        out_type=jax.ShapeDtypeStruct((total, EMB), jnp.float32),
        mesh=plsc.VectorSubcoreMesh(core_axis_name="c", subcore_axis_name="s"),
        scratch_types=(
            [pltpu.VMEM((nchunk, CH), jnp.int32),
             pltpu.VMEM((NSLOT, CH, EMB), jnp.float32)]
            + [pltpu.SemaphoreType.DMA] * (2 * NSLOT)
        ),
        compiler_params=pltpu.CompilerParams(use_tc_tiling_on_sc=False),
    )
    def k(table_hbm, idx_hbm, out_hbm, idx_v, rows_v, *sems):
        gsem = sems[:NSLOT]
        wsem = sems[NSLOT:]
        wid = lax.axis_index("s") * NC + lax.axis_index("c")
        base = wid * (nchunk * CH)
        pltpu.sync_copy(idx_hbm.at[wid], idx_v)

        def issue_gathers(j, slot):
            for k_ in range(SUB):
                sl = pl.ds(k_ * SUBROWS, SUBROWS)
                pltpu.async_copy(table_hbm.at[idx_v.at[j, sl]],
                                 rows_v.at[slot, sl], gsem[slot])

        def wait_gathers(slot):
            for k_ in range(SUB):
                sl = pl.ds(k_ * SUBROWS, SUBROWS)
                pltpu.make_async_copy(table_hbm.at[idx_v.at[0, sl]],
                                      rows_v.at[slot, sl], gsem[slot]).wait()

        def wait_write(slot):
            pltpu.make_async_copy(rows_v.at[slot],
                                  out_hbm.at[pl.ds(0, CH)],
                                  wsem[slot]).wait()

        for c in range(GAHEAD):
            issue_gathers(c, c)

        @pl.loop(0, nchunk, step=NSLOT)
        def _(g):
            for bb in range(NSLOT):
                j = g + bb
                bn = (bb + GAHEAD) % NSLOT
                wait_gathers(bb)
                pltpu.async_copy(rows_v.at[bb],
                                 out_hbm.at[pl.ds(base + j * CH, CH)],
                                 wsem[bb])

                @pl.when(j + GAHEAD < nchunk)
                def _():
                    @pl.when(j >= NSLOT - GAHEAD)
                    def _():
                        wait_write(bn)
                    issue_gathers(j + GAHEAD, bn)

        for b in range(NSLOT):
            wait_write(b)

    return k(table, idx3)


SPLIT = 499968


def _tc_body(ids_ref, lab_ref, w0_ref, w1_ref, a_ref, an_ref, l_ref, s_ref,
             b_ref, out_ref):
    bblk, s_len = ids_ref.shape
    ids = ids_ref[...]
    w0 = w0_ref[...].reshape(bblk, s_len, EMB)
    w1 = w1_ref[...].reshape(bblk, s_len, EMB)
    w = jnp.where(ids[:, :, None] < SPLIT, w0, w1)
    counts = jnp.sum((ids != 0).astype(jnp.int32), axis=1, keepdims=True)
    an_idx = counts // GROUP
    an_oh = (an_idx == lax.broadcasted_iota(jnp.int32, (bblk, 50), 1)
             ).astype(jnp.float32)
    lab_oh = (lab_ref[...] == lax.broadcasted_iota(jnp.int32, (bblk, 32), 1)
              ).astype(jnp.float32)
    c_vec = (jnp.dot(an_oh, an_ref[...], preferred_element_type=jnp.float32)
             + jnp.dot(lab_oh, l_ref[...], preferred_element_type=jnp.float32))
    s_oh = (lax.broadcasted_iota(jnp.int32, (s_len, 50), 0) // GROUP
            == lax.broadcasted_iota(jnp.int32, (s_len, 50), 1)
            ).astype(jnp.float32)
    a_vec = jnp.dot(s_oh, a_ref[...], preferred_element_type=jnp.float32)
    x = w + a_vec[None, :, :] + c_vec[:, None, :]
    mean = jnp.mean(x, axis=-1, keepdims=True)
    xc = x - mean
    var = jnp.mean(xc * xc, axis=-1, keepdims=True)
    y = xc * lax.rsqrt(var + LN_EPS)
    out_ref[...] = y * s_ref[...] + b_ref[...]


def _tc_fuse(rows0, rows1, input_ids, labels, asset_emb, asset_num_emb,
             label_emb, ln_scale, ln_bias):
    batch, s_len = input_ids.shape
    bblk = 32
    grid = (batch // bblk,)
    return pl.pallas_call(
        _tc_body,
        grid=grid,
        in_specs=[
            pl.BlockSpec((bblk, s_len), lambda i: (i, 0)),
            pl.BlockSpec((bblk, 1), lambda i: (i, 0)),
            pl.BlockSpec((bblk * s_len, EMB), lambda i: (i, 0)),
            pl.BlockSpec((bblk * s_len, EMB), lambda i: (i, 0)),
            pl.BlockSpec(asset_emb.shape, lambda i: (0, 0)),
            pl.BlockSpec(asset_num_emb.shape, lambda i: (0, 0)),
            pl.BlockSpec(label_emb.shape, lambda i: (0, 0)),
            pl.BlockSpec((1, EMB), lambda i: (0, 0)),
            pl.BlockSpec((1, EMB), lambda i: (0, 0)),
        ],
        out_specs=pl.BlockSpec((bblk, s_len, EMB), lambda i: (i, 0, 0)),
        out_shape=jax.ShapeDtypeStruct((batch, s_len, EMB), jnp.float32),
    )(input_ids, labels, rows0, rows1, asset_emb, asset_num_emb, label_emb,
      ln_scale.reshape(1, EMB), ln_bias.reshape(1, EMB))


def kernel(input_ids, labels, word_emb, asset_emb, asset_num_emb, label_emb,
           ln_scale, ln_bias, deterministic=True):
    batch, s_len = input_ids.shape
    total = batch * s_len
    nchunk = total // (NW * CH)
    flat = input_ids.reshape(-1)
    in0 = flat < SPLIT
    idx0 = jnp.where(in0, flat, jnp.minimum(flat - SPLIT, SPLIT - 1))
    idx1 = jnp.where(in0, flat, flat - SPLIT)
    part0 = word_emb[:SPLIT]
    part1 = word_emb[SPLIT:]
    rows0 = _sc_gather(part0, idx0.reshape(NW, nchunk, CH))
    rows1 = _sc_gather(part1, idx1.reshape(NW, nchunk, CH))
    return _tc_fuse(rows0, rows1, input_ids, labels, asset_emb,
                    asset_num_emb, label_emb, ln_scale, ln_bias)


# speedup vs baseline: 1.2601x; 1.2601x over previous
import functools

import jax
import jax.numpy as jnp
from jax import lax
from jax.experimental import pallas as pl
from jax.experimental.pallas import tpu as pltpu
from jax.experimental.pallas import tpu_sc as plsc

EMB = 64
GROUP = 5
LN_EPS = 1e-12

NC, NS = 2, 16
NW = NC * NS
CH = 128
SUB = 8
SUBROWS = CH // SUB
NSLOT = 5
GAHEAD = 3


def _sc_gather(table, idx3):
    nchunk = idx3.shape[1]
    total = NW * nchunk * CH

    @functools.partial(
        pl.kernel,
        out_type=jax.ShapeDtypeStruct((total, EMB), jnp.float32),
        mesh=plsc.VectorSubcoreMesh(core_axis_name="c", subcore_axis_name="s"),
        scratch_types=(
            [pltpu.VMEM((nchunk, CH), jnp.int32),
             pltpu.VMEM((NSLOT, CH, EMB), jnp.float32)]
            + [pltpu.SemaphoreType.DMA] * (2 * NSLOT)
        ),
        compiler_params=pltpu.CompilerParams(use_tc_tiling_on_sc=False),
    )
    def k(table_hbm, idx_hbm, out_hbm, idx_v, rows_v, *sems):
        gsem = sems[:NSLOT]
        wsem = sems[NSLOT:]
        wid = lax.axis_index("s") * NC + lax.axis_index("c")
        base = wid * (nchunk * CH)
        pltpu.sync_copy(idx_hbm.at[wid], idx_v)

        def issue_gathers(j, slot):
            for k_ in range(SUB):
                sl = pl.ds(k_ * SUBROWS, SUBROWS)
                pltpu.async_copy(table_hbm.at[idx_v.at[j, sl]],
                                 rows_v.at[slot, sl], gsem[slot])

        def wait_gathers(slot):
            for k_ in range(SUB):
                sl = pl.ds(k_ * SUBROWS, SUBROWS)
                pltpu.make_async_copy(table_hbm.at[idx_v.at[0, sl]],
                                      rows_v.at[slot, sl], gsem[slot]).wait()

        def wait_write(slot):
            pltpu.make_async_copy(rows_v.at[slot],
                                  out_hbm.at[pl.ds(0, CH)],
                                  wsem[slot]).wait()

        for c in range(GAHEAD):
            issue_gathers(c, c)

        @pl.loop(0, nchunk, step=NSLOT)
        def _(g):
            for bb in range(NSLOT):
                j = g + bb
                bn = (bb + GAHEAD) % NSLOT
                wait_gathers(bb)
                pltpu.async_copy(rows_v.at[bb],
                                 out_hbm.at[pl.ds(base + j * CH, CH)],
                                 wsem[bb])

                @pl.when(j + GAHEAD < nchunk)
                def _():
                    @pl.when(j >= NSLOT - GAHEAD)
                    def _():
                        wait_write(bn)
                    issue_gathers(j + GAHEAD, bn)

        for b in range(NSLOT):
            wait_write(b)

    return k(table, idx3)


def _tc_body(ids_ref, lab_ref, w_ref, a_ref, an_ref, l_ref, s_ref, b_ref,
             out_ref):
    bblk, s_len = ids_ref.shape
    ids = ids_ref[...]
    w = w_ref[...].reshape(bblk, s_len, EMB)
    counts = jnp.sum((ids != 0).astype(jnp.int32), axis=1, keepdims=True)
    an_idx = counts // GROUP
    an_oh = (an_idx == lax.broadcasted_iota(jnp.int32, (bblk, 50), 1)
             ).astype(jnp.float32)
    lab_oh = (lab_ref[...] == lax.broadcasted_iota(jnp.int32, (bblk, 32), 1)
              ).astype(jnp.float32)
    c_vec = (jnp.dot(an_oh, an_ref[...], preferred_element_type=jnp.float32)
             + jnp.dot(lab_oh, l_ref[...], preferred_element_type=jnp.float32))
    s_oh = (lax.broadcasted_iota(jnp.int32, (s_len, 50), 0) // GROUP
            == lax.broadcasted_iota(jnp.int32, (s_len, 50), 1)
            ).astype(jnp.float32)
    a_vec = jnp.dot(s_oh, a_ref[...], preferred_element_type=jnp.float32)
    x = w + a_vec[None, :, :] + c_vec[:, None, :]
    mean = jnp.mean(x, axis=-1, keepdims=True)
    xc = x - mean
    var = jnp.mean(xc * xc, axis=-1, keepdims=True)
    y = xc * lax.rsqrt(var + LN_EPS)
    out_ref[...] = y * s_ref[...] + b_ref[...]


def _tc_fuse(rows, input_ids, labels, asset_emb, asset_num_emb, label_emb,
             ln_scale, ln_bias):
    batch, s_len = input_ids.shape
    bblk = 64
    grid = (batch // bblk,)
    return pl.pallas_call(
        _tc_body,
        grid=grid,
        in_specs=[
            pl.BlockSpec((bblk, s_len), lambda i: (i, 0)),
            pl.BlockSpec((bblk, 1), lambda i: (i, 0)),
            pl.BlockSpec((bblk * s_len, EMB), lambda i: (i, 0)),
            pl.BlockSpec(asset_emb.shape, lambda i: (0, 0)),
            pl.BlockSpec(asset_num_emb.shape, lambda i: (0, 0)),
            pl.BlockSpec(label_emb.shape, lambda i: (0, 0)),
            pl.BlockSpec((1, EMB), lambda i: (0, 0)),
            pl.BlockSpec((1, EMB), lambda i: (0, 0)),
        ],
        out_specs=pl.BlockSpec((bblk, s_len, EMB), lambda i: (i, 0, 0)),
        out_shape=jax.ShapeDtypeStruct((batch, s_len, EMB), jnp.float32),
    )(input_ids, labels, rows, asset_emb, asset_num_emb, label_emb,
      ln_scale.reshape(1, EMB), ln_bias.reshape(1, EMB))


def kernel(input_ids, labels, word_emb, asset_emb, asset_num_emb, label_emb,
           ln_scale, ln_bias, deterministic=True):
    batch, s_len = input_ids.shape
    total = batch * s_len
    nchunk = total // (NW * CH)
    idx3 = input_ids.reshape(NW, nchunk, CH)
    rows = _sc_gather(word_emb, idx3)
    return _tc_fuse(rows, input_ids, labels, asset_emb, asset_num_emb,
                    label_emb, ln_scale, ln_bias)
